# baseline (speedup 1.0000x reference)
import jax
import jax.numpy as jnp
from jax import lax
from jax.experimental import pallas as pl
from jax.experimental.pallas import tpu as pltpu
from jax.experimental.pallas import tpu_sc as plsc

B = 16384
E = 64
NUM_ROWS = 100000
NC, NS, L = 2, 16, 16
NW = NC * NS
BPW = B // NW
CH = 128
NCH = BPW // CH

_MESH = plsc.VectorSubcoreMesh(
    core_axis_name="c", subcore_axis_name="s", num_cores=NC, num_subcores=NS)


def _sc_index_bias(pairs_hbm, ub_hbm, pb_hbm,
                   idxu_hbm, idxp_hbm, bsum_hbm,
                   pairs_v, idxu_v, idxp_v, ubv, pbv, bs_v, sem_b):
    wid = lax.axis_index("s") * NC + lax.axis_index("c")
    base = wid * BPW

    pltpu.sync_copy(pairs_hbm.at[pl.ds(base * 2, BPW * 2)], pairs_v)

    for n in range(BPW // L):
        flat = (lax.iota(jnp.int32, L) + n * L) * 2
        j, k = divmod(n, CH // L)
        idxu_v[j, pl.ds(k * L, L)] = plsc.load_gather(pairs_v, [flat])
        idxp_v[j, pl.ds(k * L, L)] = plsc.load_gather(pairs_v, [flat + 1])

    handles = []
    for j in range(NCH):
        handles.append(pltpu.async_copy(
            ub_hbm.at[idxu_v.at[j]], ubv.at[j], sem_b))
        handles.append(pltpu.async_copy(
            pb_hbm.at[idxp_v.at[j]], pbv.at[j], sem_b))
    for h in handles:
        h.wait()

    for j in range(NCH):
        for k in range(CH // L):
            s = ubv[j, pl.ds(k * L, L)] + pbv[j, pl.ds(k * L, L)]
            bs_v[pl.ds(j * CH + k * L, L)] = s
    pltpu.sync_copy(bs_v, bsum_hbm.at[pl.ds(base, BPW)])

    for j in range(NCH):
        pltpu.sync_copy(idxu_v.at[j], idxu_hbm.at[pl.ds(base + j * CH, CH)])
        pltpu.sync_copy(idxp_v.at[j], idxp_hbm.at[pl.ds(base + j * CH, CH)])


RCH = 256
NRCH = BPW // RCH


def _sc_row_dot(idxu_hbm, idxp_hbm, uemb_hbm, pemb_hbm,
                partials_hbm,
                idxu_v, idxp_v, urows_v, prows_v, part_v, sem_u, sem_p):
    wid = lax.axis_index("s") * NC + lax.axis_index("c")
    base = wid * BPW

    pltpu.sync_copy(idxu_hbm.at[pl.ds(base, BPW)], idxu_v)
    pltpu.sync_copy(idxp_hbm.at[pl.ds(base, BPW)], idxp_v)

    acc = jnp.zeros((L,), jnp.float32)
    for ch in range(NRCH):
        def enq(n, carry, ch=ch):
            uvec = idxu_v[pl.ds(ch * RCH + n * L, L)]
            pvec = idxp_v[pl.ds(ch * RCH + n * L, L)]
            for l in range(L):
                r = n * L + l
                pltpu.async_copy(uemb_hbm.at[pl.ds(uvec[l], 1), :],
                                 urows_v.at[pl.ds(r, 1), :], sem_u)
                pltpu.async_copy(pemb_hbm.at[pl.ds(pvec[l], 1), :],
                                 prows_v.at[pl.ds(r, 1), :], sem_p)
            return carry
        lax.fori_loop(0, RCH // L, enq, 0)

        def drain(n, carry):
            pltpu.make_async_copy(uemb_hbm.at[pl.ds(0, 1), :],
                                  urows_v.at[pl.ds(0, 1), :], sem_u).wait()
            pltpu.make_async_copy(pemb_hbm.at[pl.ds(0, 1), :],
                                  prows_v.at[pl.ds(0, 1), :], sem_p).wait()
            return carry
        lax.fori_loop(0, RCH, drain, 0)

        def dot_body(n, a):
            for c in range(E // L):
                a = a + (urows_v[n, pl.ds(c * L, L)]
                         * prows_v[n, pl.ds(c * L, L)])
            return a
        acc = lax.fori_loop(0, RCH, dot_body, acc)
    part_v[...] = acc
    pltpu.sync_copy(part_v, partials_hbm.at[pl.ds(wid * L, L)])


TBK = 1024


def _tc_transpose(uT_ref, pT_ref, u_out_ref, p_out_ref):
    i = lax.broadcasted_iota(jnp.int32, (E, E), 0)
    j = lax.broadcasted_iota(jnp.int32, (E, E), 1)
    eye = (i == j).astype(jnp.float32)
    dims = (((0,), (0,)), ((), ()))
    u_out_ref[...] = lax.dot_general(uT_ref[...], eye, dims,
                                     preferred_element_type=jnp.float32)
    p_out_ref[...] = lax.dot_general(pT_ref[...], eye, dims,
                                     preferred_element_type=jnp.float32)


def _tc_finish(partials_ref, bsum_ref, out_ref):
    total = jnp.sum(partials_ref[...])
    x = total + bsum_ref[...]
    out_ref[...] = 1.0 / (1.0 + jnp.exp(-x))


def kernel(inputs, user_embedding, user_bias_table, place_embedding,
           place_bias_table):
    pairs = inputs.astype(jnp.int32).reshape(-1)
    ub = user_bias_table.reshape(-1)
    pb = place_bias_table.reshape(-1)

    sc_a = pl.kernel(
        _sc_index_bias,
        out_type=[
            jax.ShapeDtypeStruct((B,), jnp.int32),
            jax.ShapeDtypeStruct((B,), jnp.int32),
            jax.ShapeDtypeStruct((B,), jnp.float32),
        ],
        mesh=_MESH,
        compiler_params=pltpu.CompilerParams(
            needs_layout_passes=False, use_tc_tiling_on_sc=False),
        scratch_types=[
            pltpu.VMEM((BPW * 2,), jnp.int32),
            pltpu.VMEM((NCH, CH), jnp.int32),
            pltpu.VMEM((NCH, CH), jnp.int32),
            pltpu.VMEM((NCH, CH), jnp.float32),
            pltpu.VMEM((NCH, CH), jnp.float32),
            pltpu.VMEM((BPW,), jnp.float32),
            pltpu.SemaphoreType.DMA,
        ],
    )
    idxu, idxp, bsum = sc_a(pairs, ub, pb)

    uemb_t, pemb_t = pl.pallas_call(
        _tc_transpose,
        grid=(pl.cdiv(NUM_ROWS, TBK),),
        in_specs=[
            pl.BlockSpec((E, TBK), lambda i: (0, i)),
            pl.BlockSpec((E, TBK), lambda i: (0, i)),
        ],
        out_specs=[
            pl.BlockSpec((TBK, E), lambda i: (i, 0)),
            pl.BlockSpec((TBK, E), lambda i: (i, 0)),
        ],
        out_shape=[
            jax.ShapeDtypeStruct((NUM_ROWS, E), jnp.float32),
            jax.ShapeDtypeStruct((NUM_ROWS, E), jnp.float32),
        ],
    )(user_embedding.T, place_embedding.T)

    sc_b = pl.kernel(
        _sc_row_dot,
        out_type=[jax.ShapeDtypeStruct((NW * L,), jnp.float32)],
        mesh=_MESH,
        compiler_params=pltpu.CompilerParams(
            needs_layout_passes=False, use_tc_tiling_on_sc=True),
        scratch_types=[
            pltpu.VMEM((BPW,), jnp.int32),
            pltpu.VMEM((BPW,), jnp.int32),
            pltpu.VMEM((RCH, E), jnp.float32),
            pltpu.VMEM((RCH, E), jnp.float32),
            pltpu.VMEM((L,), jnp.float32),
            pltpu.SemaphoreType.DMA,
            pltpu.SemaphoreType.DMA,
        ],
    )
    (partials,) = sc_b(idxu, idxp, uemb_t, pemb_t)

    out = pl.pallas_call(
        _tc_finish,
        out_shape=jax.ShapeDtypeStruct((B // 128, 128), jnp.float32),
    )(partials.reshape(NW * L // 128, 128), bsum.reshape(B // 128, 128))
    return out.reshape(B, 1)


# speedup vs baseline: 1.0362x; 1.0362x over previous
import jax
import jax.numpy as jnp
from jax import lax
from jax.experimental import pallas as pl
from jax.experimental.pallas import tpu as pltpu
from jax.experimental.pallas import tpu_sc as plsc

B = 16384
E = 64
NUM_ROWS = 100000
NC, NS, L = 2, 16, 16
NW = NC * NS
BPW = B // NW
CH = 128
NCH = BPW // CH

_MESH = plsc.VectorSubcoreMesh(
    core_axis_name="c", subcore_axis_name="s", num_cores=NC, num_subcores=NS)


def _sc_index_bias(pairs_hbm, ub_hbm, pb_hbm,
                   idxu_hbm, idxp_hbm, bsum_hbm,
                   pairs_v, idxu_v, idxp_v, ubv, pbv, bs_v, sem_b):
    wid = lax.axis_index("s") * NC + lax.axis_index("c")
    base = wid * BPW

    pltpu.sync_copy(pairs_hbm.at[pl.ds(base * 2, BPW * 2)], pairs_v)

    for n in range(BPW // L):
        flat = (lax.iota(jnp.int32, L) + n * L) * 2
        j, k = divmod(n, CH // L)
        idxu_v[j, pl.ds(k * L, L)] = plsc.load_gather(pairs_v, [flat])
        idxp_v[j, pl.ds(k * L, L)] = plsc.load_gather(pairs_v, [flat + 1])

    handles = []
    for j in range(NCH):
        handles.append(pltpu.async_copy(
            ub_hbm.at[idxu_v.at[j]], ubv.at[j], sem_b))
        handles.append(pltpu.async_copy(
            pb_hbm.at[idxp_v.at[j]], pbv.at[j], sem_b))
    for h in handles:
        h.wait()

    for j in range(NCH):
        for k in range(CH // L):
            s = ubv[j, pl.ds(k * L, L)] + pbv[j, pl.ds(k * L, L)]
            bs_v[pl.ds(j * CH + k * L, L)] = s
    pltpu.sync_copy(bs_v, bsum_hbm.at[pl.ds(base, BPW)])

    for j in range(NCH):
        pltpu.sync_copy(idxu_v.at[j], idxu_hbm.at[pl.ds(base + j * CH, CH)])
        pltpu.sync_copy(idxp_v.at[j], idxp_hbm.at[pl.ds(base + j * CH, CH)])


RCH = 256
NRCH = BPW // RCH


def _sc_row_dot(idxu_hbm, idxp_hbm, uemb_hbm, pemb_hbm,
                partials_hbm,
                idxu_v, idxp_v, urows_v, prows_v, part_v, sem_u, sem_p):
    wid = lax.axis_index("s") * NC + lax.axis_index("c")
    base = wid * BPW

    pltpu.sync_copy(idxu_hbm.at[pl.ds(base, BPW)], idxu_v)
    pltpu.sync_copy(idxp_hbm.at[pl.ds(base, BPW)], idxp_v)

    acc = jnp.zeros((L,), jnp.float32)
    for ch in range(NRCH):
        def enq(n, carry, ch=ch):
            uvec = idxu_v[pl.ds(ch * RCH + n * L, L)]
            pvec = idxp_v[pl.ds(ch * RCH + n * L, L)]
            for l in range(L):
                r = n * L + l
                pltpu.async_copy(uemb_hbm.at[pl.ds(uvec[l], 1), :],
                                 urows_v.at[pl.ds(r, 1), :], sem_u)
                pltpu.async_copy(pemb_hbm.at[pl.ds(pvec[l], 1), :],
                                 prows_v.at[pl.ds(r, 1), :], sem_p)
            return carry
        lax.fori_loop(0, RCH // L, enq, 0)

        def drain(n, carry):
            pltpu.make_async_copy(uemb_hbm.at[pl.ds(0, 1), :],
                                  urows_v.at[pl.ds(0, 1), :], sem_u).wait()
            pltpu.make_async_copy(pemb_hbm.at[pl.ds(0, 1), :],
                                  prows_v.at[pl.ds(0, 1), :], sem_p).wait()
            return carry
        lax.fori_loop(0, RCH, drain, 0)

        def dot_body(n, a):
            for c in range(E // L):
                a = a + (urows_v[n, pl.ds(c * L, L)]
                         * prows_v[n, pl.ds(c * L, L)])
            return a
        acc = lax.fori_loop(0, RCH, dot_body, acc)
    part_v[...] = acc
    pltpu.sync_copy(part_v, partials_hbm.at[pl.ds(wid * L, L)])


TBK = 1024


def _tc_transpose(uT_ref, pT_ref, u_out_ref, p_out_ref):
    u_out_ref[...] = uT_ref[...].T
    p_out_ref[...] = pT_ref[...].T


def _tc_finish(partials_ref, bsum_ref, out_ref):
    total = jnp.sum(partials_ref[...])
    x = total + bsum_ref[...]
    out_ref[...] = 1.0 / (1.0 + jnp.exp(-x))


def kernel(inputs, user_embedding, user_bias_table, place_embedding,
           place_bias_table):
    pairs = inputs.astype(jnp.int32).reshape(-1)
    ub = user_bias_table.reshape(-1)
    pb = place_bias_table.reshape(-1)

    sc_a = pl.kernel(
        _sc_index_bias,
        out_type=[
            jax.ShapeDtypeStruct((B,), jnp.int32),
            jax.ShapeDtypeStruct((B,), jnp.int32),
            jax.ShapeDtypeStruct((B,), jnp.float32),
        ],
        mesh=_MESH,
        compiler_params=pltpu.CompilerParams(
            needs_layout_passes=False, use_tc_tiling_on_sc=False),
        scratch_types=[
            pltpu.VMEM((BPW * 2,), jnp.int32),
            pltpu.VMEM((NCH, CH), jnp.int32),
            pltpu.VMEM((NCH, CH), jnp.int32),
            pltpu.VMEM((NCH, CH), jnp.float32),
            pltpu.VMEM((NCH, CH), jnp.float32),
            pltpu.VMEM((BPW,), jnp.float32),
            pltpu.SemaphoreType.DMA,
        ],
    )
    idxu, idxp, bsum = sc_a(pairs, ub, pb)

    uemb_t, pemb_t = pl.pallas_call(
        _tc_transpose,
        grid=(pl.cdiv(NUM_ROWS, TBK),),
        in_specs=[
            pl.BlockSpec((E, TBK), lambda i: (0, i)),
            pl.BlockSpec((E, TBK), lambda i: (0, i)),
        ],
        out_specs=[
            pl.BlockSpec((TBK, E), lambda i: (i, 0)),
            pl.BlockSpec((TBK, E), lambda i: (i, 0)),
        ],
        out_shape=[
            jax.ShapeDtypeStruct((NUM_ROWS, E), jnp.float32),
            jax.ShapeDtypeStruct((NUM_ROWS, E), jnp.float32),
        ],
        compiler_params=pltpu.CompilerParams(
            fuse_transposed_lhs_in_matmul=True),
    )(user_embedding.T, place_embedding.T)

    sc_b = pl.kernel(
        _sc_row_dot,
        out_type=[jax.ShapeDtypeStruct((NW * L,), jnp.float32)],
        mesh=_MESH,
        compiler_params=pltpu.CompilerParams(
            needs_layout_passes=False, use_tc_tiling_on_sc=True),
        scratch_types=[
            pltpu.VMEM((BPW,), jnp.int32),
            pltpu.VMEM((BPW,), jnp.int32),
            pltpu.VMEM((RCH, E), jnp.float32),
            pltpu.VMEM((RCH, E), jnp.float32),
            pltpu.VMEM((L,), jnp.float32),
            pltpu.SemaphoreType.DMA,
            pltpu.SemaphoreType.DMA,
        ],
    )
    (partials,) = sc_b(idxu, idxp, uemb_t, pemb_t)

    out = pl.pallas_call(
        _tc_finish,
        out_shape=jax.ShapeDtypeStruct((B // 128, 128), jnp.float32),
    )(partials.reshape(NW * L // 128, 128), bsum.reshape(B // 128, 128))
    return out.reshape(B, 1)


# speedup vs baseline: 1.2171x; 1.1746x over previous
import jax
import jax.numpy as jnp
from jax import lax
from jax.experimental import pallas as pl
from jax.experimental.pallas import tpu as pltpu
from jax.experimental.pallas import tpu_sc as plsc

B = 16384
E = 64
NUM_ROWS = 100000
NC, NS, L = 2, 16, 16
NW = NC * NS
BPW = B // NW
CH = 128
NCH = BPW // CH

_MESH = plsc.VectorSubcoreMesh(
    core_axis_name="c", subcore_axis_name="s", num_cores=NC, num_subcores=NS)


def _sc_index_bias(pairs_hbm, ub_hbm, pb_hbm,
                   idxu_hbm, idxp_hbm, bsum_hbm,
                   pairs_v, idxu_v, idxp_v, ubv, pbv, bs_v, sem_b):
    wid = lax.axis_index("s") * NC + lax.axis_index("c")
    base = wid * BPW

    pltpu.sync_copy(pairs_hbm.at[pl.ds(base * 2, BPW * 2)], pairs_v)

    for n in range(BPW // L):
        flat = (lax.iota(jnp.int32, L) + n * L) * 2
        j, k = divmod(n, CH // L)
        idxu_v[j, pl.ds(k * L, L)] = plsc.load_gather(pairs_v, [flat])
        idxp_v[j, pl.ds(k * L, L)] = plsc.load_gather(pairs_v, [flat + 1])

    handles = []
    for j in range(NCH):
        handles.append(pltpu.async_copy(
            ub_hbm.at[idxu_v.at[j]], ubv.at[j], sem_b))
        handles.append(pltpu.async_copy(
            pb_hbm.at[idxp_v.at[j]], pbv.at[j], sem_b))
    for h in handles:
        h.wait()

    for j in range(NCH):
        for k in range(CH // L):
            s = ubv[j, pl.ds(k * L, L)] + pbv[j, pl.ds(k * L, L)]
            bs_v[pl.ds(j * CH + k * L, L)] = s
    pltpu.sync_copy(bs_v, bsum_hbm.at[pl.ds(base, BPW)])

    for j in range(NCH):
        pltpu.sync_copy(idxu_v.at[j], idxu_hbm.at[pl.ds(base + j * CH, CH)])
        pltpu.sync_copy(idxp_v.at[j], idxp_hbm.at[pl.ds(base + j * CH, CH)])


RCH = 256
NRCH = BPW // RCH


def _sc_row_dot(idxu_hbm, idxp_hbm, uemb_hbm, pemb_hbm,
                partials_hbm,
                idxu_v, idxp_v, urows_v, prows_v, part_v, sem_u, sem_p):
    wid = lax.axis_index("s") * NC + lax.axis_index("c")
    base = wid * BPW

    pltpu.sync_copy(idxu_hbm.at[pl.ds(base, BPW)], idxu_v)
    pltpu.sync_copy(idxp_hbm.at[pl.ds(base, BPW)], idxp_v)

    acc = jnp.zeros((L,), jnp.float32)
    for ch in range(NRCH):
        def enq(n, carry, ch=ch):
            uvec = idxu_v[pl.ds(ch * RCH + n * L, L)]
            pvec = idxp_v[pl.ds(ch * RCH + n * L, L)]
            for l in range(L):
                r = n * L + l
                pltpu.async_copy(uemb_hbm.at[pl.ds(uvec[l], 1), :],
                                 urows_v.at[pl.ds(r, 1), :], sem_u)
                pltpu.async_copy(pemb_hbm.at[pl.ds(pvec[l], 1), :],
                                 prows_v.at[pl.ds(r, 1), :], sem_p)
            return carry
        lax.fori_loop(0, RCH // L, enq, 0)

        def drain(n, carry):
            pltpu.make_async_copy(uemb_hbm.at[pl.ds(0, 1), :],
                                  urows_v.at[pl.ds(0, 1), :], sem_u).wait()
            pltpu.make_async_copy(pemb_hbm.at[pl.ds(0, 1), :],
                                  prows_v.at[pl.ds(0, 1), :], sem_p).wait()
            return carry
        lax.fori_loop(0, RCH, drain, 0)

        def dot_body(n, a):
            for c in range(E // L):
                a = a + (urows_v[n, pl.ds(c * L, L)]
                         * prows_v[n, pl.ds(c * L, L)])
            return a
        acc = lax.fori_loop(0, RCH, dot_body, acc)
    part_v[...] = acc
    pltpu.sync_copy(part_v, partials_hbm.at[pl.ds(wid * L, L)])


def _tc_finish(partials_ref, bsum_ref, out_ref):
    total = jnp.sum(partials_ref[...])
    x = total + bsum_ref[...]
    out_ref[...] = 1.0 / (1.0 + jnp.exp(-x))


def kernel(inputs, user_embedding, user_bias_table, place_embedding,
           place_bias_table):
    pairs = inputs.astype(jnp.int32).reshape(-1)
    ub = user_bias_table.reshape(-1)
    pb = place_bias_table.reshape(-1)

    sc_a = pl.kernel(
        _sc_index_bias,
        out_type=[
            jax.ShapeDtypeStruct((B,), jnp.int32),
            jax.ShapeDtypeStruct((B,), jnp.int32),
            jax.ShapeDtypeStruct((B,), jnp.float32),
        ],
        mesh=_MESH,
        compiler_params=pltpu.CompilerParams(
            needs_layout_passes=False, use_tc_tiling_on_sc=False),
        scratch_types=[
            pltpu.VMEM((BPW * 2,), jnp.int32),
            pltpu.VMEM((NCH, CH), jnp.int32),
            pltpu.VMEM((NCH, CH), jnp.int32),
            pltpu.VMEM((NCH, CH), jnp.float32),
            pltpu.VMEM((NCH, CH), jnp.float32),
            pltpu.VMEM((BPW,), jnp.float32),
            pltpu.SemaphoreType.DMA,
        ],
    )
    idxu, idxp, bsum = sc_a(pairs, ub, pb)

    sc_b = pl.kernel(
        _sc_row_dot,
        out_type=[jax.ShapeDtypeStruct((NW * L,), jnp.float32)],
        mesh=_MESH,
        compiler_params=pltpu.CompilerParams(
            needs_layout_passes=False, use_tc_tiling_on_sc=True),
        scratch_types=[
            pltpu.VMEM((BPW,), jnp.int32),
            pltpu.VMEM((BPW,), jnp.int32),
            pltpu.VMEM((RCH, E), jnp.float32),
            pltpu.VMEM((RCH, E), jnp.float32),
            pltpu.VMEM((L,), jnp.float32),
            pltpu.SemaphoreType.DMA,
            pltpu.SemaphoreType.DMA,
        ],
    )
    (partials,) = sc_b(idxu, idxp, user_embedding, place_embedding)

    out = pl.pallas_call(
        _tc_finish,
        out_shape=jax.ShapeDtypeStruct((B // 128, 128), jnp.float32),
    )(partials.reshape(NW * L // 128, 128), bsum.reshape(B // 128, 128))
    return out.reshape(B, 1)
